# baseline (speedup 1.0000x reference)
import functools

import jax
import jax.numpy as jnp
from jax import lax
from jax.experimental import pallas as pl
from jax.experimental.pallas import tpu as pltpu
from jax.experimental.pallas import tpu_sc as plsc

NC = 2
NS = 16
NPAD = 10240
NPADE = 10048


def _scalar_propagate(vals, ei):
  n = vals.shape[0] if vals is not None else 0
  e = ei.shape[1]
  e_w = e // (NC * NS)
  ch = 2000
  n_ch = e_w // ch
  unroll = 5
  slc = NPAD // NS
  mesh = plsc.VectorSubcoreMesh(core_axis_name="c", subcore_axis_name="s")

  vals_scratch = [pltpu.VMEM((n,), jnp.float32)] if vals is not None else []

  @functools.partial(
      pl.kernel,
      mesh=mesh,
      compiler_params=pltpu.CompilerParams(needs_layout_passes=False,
                                           use_tc_tiling_on_sc=False),
      out_type=jax.ShapeDtypeStruct((NC, NPAD), jnp.float32),
      scratch_types=vals_scratch + [
          pltpu.VMEM((NPAD,), jnp.float32),
          pltpu.VMEM((ch,), jnp.int32),
          pltpu.VMEM((ch,), jnp.int32),
          pltpu.VMEM((slc,), jnp.float32),
          pltpu.VMEM((slc,), jnp.float32),
          pltpu.VMEM_SHARED((NS, NPAD), jnp.float32),
      ],
  )
  def k(*refs):
    if vals is not None:
      (vals_hbm, ei_hbm, out_hbm,
       vals_v, acc_v, src_v, dst_v, tmp_v, tot_v, shared) = refs
    else:
      (ei_hbm, out_hbm,
       acc_v, src_v, dst_v, tmp_v, tot_v, shared) = refs
    cid = lax.axis_index("c")
    sid = lax.axis_index("s")
    base = (cid * NS + sid) * e_w
    if vals is not None:
      pltpu.sync_copy(vals_hbm, vals_v)

    zero16 = jnp.zeros((16,), jnp.float32)
    ones16 = jnp.ones((16,), jnp.float32)

    def zero_body(i, _):
      acc_v[pl.ds(i * 16, 16)] = zero16
      return 0
    lax.fori_loop(0, NPAD // 16, zero_body, 0)

    def chunk_body(c, _):
      off = base + c * ch
      pltpu.sync_copy(ei_hbm.at[0, pl.ds(off, ch)], src_v)
      pltpu.sync_copy(ei_hbm.at[1, pl.ds(off, ch)], dst_v)

      def edge_body(j, _):
        for u in range(unroll):
          p = (j * unroll + u) * 16
          d16 = dst_v[pl.ds(p, 16)]
          if vals is not None:
            s16 = src_v[pl.ds(p, 16)]
            v16 = plsc.load_gather(vals_v, [s16])
          else:
            v16 = ones16
          plsc.addupdate_scatter(acc_v, [d16], v16)
        return 0
      lax.fori_loop(0, ch // (16 * unroll), edge_body, 0)
      return 0
    lax.fori_loop(0, n_ch, chunk_body, 0)

    pltpu.sync_copy(acc_v, shared.at[sid])
    plsc.subcore_barrier()

    def zt_body(i, _):
      tot_v[pl.ds(i * 16, 16)] = zero16
      return 0
    lax.fori_loop(0, slc // 16, zt_body, 0)

    for j in range(NS):
      pltpu.sync_copy(shared.at[j, pl.ds(sid * slc, slc)], tmp_v)

      def add_body(i, _):
        tot_v[pl.ds(i * 16, 16)] += tmp_v[pl.ds(i * 16, 16)]
        return 0
      lax.fori_loop(0, slc // 16, add_body, 0)

    pltpu.sync_copy(tot_v, out_hbm.at[cid, pl.ds(sid * slc, slc)])

  if vals is not None:
    return k(vals, ei)
  return k(ei)


def _edge_propagate(g, ei4):
  n, d = g.shape
  _, nw, n_ch, kk = ei4.shape
  slc = NPADE // NS
  mesh = plsc.VectorSubcoreMesh(core_axis_name="c", subcore_axis_name="s")

  @functools.partial(
      pl.kernel,
      mesh=mesh,
      compiler_params=pltpu.CompilerParams(needs_layout_passes=False,
                                           use_tc_tiling_on_sc=False),
      out_type=jax.ShapeDtypeStruct((NC, NPADE, d), jnp.float32),
      scratch_types=[
          pltpu.VMEM((n_ch, kk), jnp.int32),
          pltpu.VMEM((n_ch, kk), jnp.int32),
          [pltpu.VMEM((kk, d), jnp.float32)] * 2,
          pltpu.VMEM_SHARED((NPADE, d), jnp.float32),
          [pltpu.SemaphoreType.DMA] * 2,
          [pltpu.SemaphoreType.DMA] * 2,
      ],
  )
  def k(g_hbm, ei_hbm, out_hbm, src_v, dst_v, rows, s_sh, sems, ssems):
    cid = lax.axis_index("c")
    sid = lax.axis_index("s")
    wid = cid * NS + sid
    pltpu.sync_copy(ei_hbm.at[0, wid], src_v)
    pltpu.sync_copy(ei_hbm.at[1, wid], dst_v)

    zero16 = jnp.zeros((16,), jnp.float32)

    def zr_body(i, _):
      rows[0][i // (d // 16), pl.ds((i % (d // 16)) * 16, 16)] = zero16
      return 0
    lax.fori_loop(0, kk * (d // 16), zr_body, 0)
    for z in range(slc // kk):
      pltpu.sync_copy(rows[0], s_sh.at[pl.ds(sid * slc + z * kk, kk)])
    rem = slc % kk
    if rem:
      pltpu.sync_copy(rows[0].at[pl.ds(0, rem)],
                      s_sh.at[pl.ds(sid * slc + (slc // kk) * kk, rem)])
    plsc.subcore_barrier()

    pltpu.async_copy(g_hbm.at[src_v.at[0]], rows[0], sems[0])
    pltpu.async_copy(g_hbm.at[src_v.at[1]], rows[1], sems[1])

    def pair_body(q, _):
      c0 = q * 2
      pltpu.make_async_copy(g_hbm.at[src_v.at[c0]], rows[0], sems[0]).wait()
      pltpu.async_copy(rows[0], s_sh.at[dst_v.at[c0]], ssems[0], add=True)
      pltpu.make_async_copy(g_hbm.at[src_v.at[c0]], rows[1], sems[1]).wait()
      pltpu.async_copy(rows[1], s_sh.at[dst_v.at[c0 + 1]], ssems[1], add=True)

      pltpu.make_async_copy(rows[0], s_sh.at[dst_v.at[c0]], ssems[0]).wait()

      @pl.when(c0 + 2 < n_ch)
      def _():
        pltpu.async_copy(g_hbm.at[src_v.at[c0 + 2]], rows[0], sems[0])

      pltpu.make_async_copy(rows[1], s_sh.at[dst_v.at[c0]], ssems[1]).wait()

      @pl.when(c0 + 3 < n_ch)
      def _():
        pltpu.async_copy(g_hbm.at[src_v.at[c0 + 3]], rows[1], sems[1])
      return 0
    lax.fori_loop(0, n_ch // 2, pair_body, 0)

    plsc.subcore_barrier()
    pltpu.sync_copy(s_sh.at[pl.ds(sid * slc, slc)],
                    out_hbm.at[cid, pl.ds(sid * slc, slc)])

  return k(g, ei4)


def _tc_matmul(x, w1):
  n, d_in = x.shape
  h = w1.shape[1]
  r = n

  def body(x_r, w_r, o_r):
    o_r[...] = jnp.dot(x_r[...], w_r[...], preferred_element_type=jnp.float32)

  return pl.pallas_call(
      body,
      grid=(n // r,),
      in_specs=[
          pl.BlockSpec((r, d_in), lambda i: (i, 0)),
          pl.BlockSpec((d_in, h), lambda i: (0, 0)),
      ],
      out_specs=pl.BlockSpec((r, h), lambda i: (i, 0)),
      out_shape=jax.ShapeDtypeStruct((n, h), jnp.float32),
  )(x, w1)


def _tc_scale1(cnt_a, cnt_b, h1):
  n, h = h1.shape
  r = n

  def body(ca_r, cb_r, h_r, d_r, g_r):
    deg = ca_r[...] + cb_r[...] + 1.0
    dv = lax.rsqrt(deg)
    d_r[...] = dv
    g_r[...] = h_r[...] * dv

  return pl.pallas_call(
      body,
      grid=(n // r,),
      in_specs=[
          pl.BlockSpec((r, 1), lambda i: (i, 0)),
          pl.BlockSpec((r, 1), lambda i: (i, 0)),
          pl.BlockSpec((r, h), lambda i: (i, 0)),
      ],
      out_specs=[
          pl.BlockSpec((r, 1), lambda i: (i, 0)),
          pl.BlockSpec((r, h), lambda i: (i, 0)),
      ],
      out_shape=[
          jax.ShapeDtypeStruct((n, 1), jnp.float32),
          jax.ShapeDtypeStruct((n, h), jnp.float32),
      ],
  )(cnt_a, cnt_b, h1)


def _tc_mid(s, g_prev, d, b, w):
  n, h_in = g_prev.shape
  h_out = w.shape[1]
  r = n

  def body(sa_r, sb_r, g_r, d_r, b_r, w_r, o_r):
    hid = jnp.maximum(
        d_r[...] * (sa_r[0] + sb_r[0] + g_r[...]) + b_r[...], 0.0)
    o_r[...] = jnp.dot(hid, w_r[...],
                       preferred_element_type=jnp.float32) * d_r[...]

  return pl.pallas_call(
      body,
      grid=(n // r,),
      in_specs=[
          pl.BlockSpec((1, r, h_in), lambda i: (0, i, 0)),
          pl.BlockSpec((1, r, h_in), lambda i: (1, i, 0)),
          pl.BlockSpec((r, h_in), lambda i: (i, 0)),
          pl.BlockSpec((r, 1), lambda i: (i, 0)),
          pl.BlockSpec((1, h_in), lambda i: (0, 0)),
          pl.BlockSpec((h_in, h_out), lambda i: (0, 0)),
      ],
      out_specs=pl.BlockSpec((r, h_out), lambda i: (i, 0)),
      out_shape=jax.ShapeDtypeStruct((n, h_out), jnp.float32),
  )(s, s, g_prev, d, b, w)


def _tc_final(s_a, s_b, g3, d, b3):
  n = g3.shape[0]

  def body(sa_r, sb_r, g_r, d_r, b_r, o_r):
    o_r[...] = d_r[...] * (sa_r[...] + sb_r[...] + g_r[...]) + b_r[...]

  return pl.pallas_call(
      body,
      grid=(1,),
      in_specs=[pl.BlockSpec((n, 1), lambda i: (0, 0))] * 4
      + [pl.BlockSpec((1, 1), lambda i: (0, 0))],
      out_specs=pl.BlockSpec((n, 1), lambda i: (0, 0)),
      out_shape=jax.ShapeDtypeStruct((n, 1), jnp.float32),
  )(s_a, s_b, g3, d, b3)


def kernel(x, edge_index, W1, b1, W2, b2, W3, b3):
  n = x.shape[0]
  e = edge_index.shape[1]
  kk = 100
  n_ch = e // (NC * NS * kk)
  ei4 = edge_index.reshape(2, NC * NS, n_ch, kk)

  cnt = _scalar_propagate(None, edge_index)
  h1 = _tc_matmul(x, W1)
  cnt_a = cnt[0, :n].reshape(n, 1)
  cnt_b = cnt[1, :n].reshape(n, 1)

  d, g1 = _tc_scale1(cnt_a, cnt_b, h1)
  s1 = _edge_propagate(g1, ei4)
  g2 = _tc_mid(s1, g1, d, b1.reshape(1, -1), W2)
  s2 = _edge_propagate(g2, ei4)
  g3 = _tc_mid(s2, g2, d, b2.reshape(1, -1), W3)
  s3 = _scalar_propagate(g3.reshape(n), edge_index)
  out = _tc_final(s3[0, :n].reshape(n, 1), s3[1, :n].reshape(n, 1),
                  g3, d, b3.reshape(1, 1))
  return out


# speedup vs baseline: 1.2085x; 1.2085x over previous
import functools

import jax
import jax.numpy as jnp
from jax import lax
from jax.experimental import pallas as pl
from jax.experimental.pallas import tpu as pltpu
from jax.experimental.pallas import tpu_sc as plsc

NC = 2
NS = 16
NPAD = 10240
NPADE = 10048


def _scalar_propagate(vals, ei):
  n = vals.shape[0] if vals is not None else 0
  e = ei.shape[1]
  e_w = e // (NC * NS)
  ch = 2000
  n_ch = e_w // ch
  unroll = 5
  slc = NPAD // NS
  mesh = plsc.VectorSubcoreMesh(core_axis_name="c", subcore_axis_name="s")

  vals_scratch = [pltpu.VMEM((n,), jnp.float32)] if vals is not None else []

  @functools.partial(
      pl.kernel,
      mesh=mesh,
      compiler_params=pltpu.CompilerParams(needs_layout_passes=False,
                                           use_tc_tiling_on_sc=False),
      out_type=jax.ShapeDtypeStruct((NC, NPAD), jnp.float32),
      scratch_types=vals_scratch + [
          pltpu.VMEM((NPAD,), jnp.float32),
          pltpu.VMEM((ch,), jnp.int32),
          pltpu.VMEM((ch,), jnp.int32),
          pltpu.VMEM((slc,), jnp.float32),
          pltpu.VMEM((slc,), jnp.float32),
          pltpu.VMEM_SHARED((NS, NPAD), jnp.float32),
      ],
  )
  def k(*refs):
    if vals is not None:
      (vals_hbm, ei_hbm, out_hbm,
       vals_v, acc_v, src_v, dst_v, tmp_v, tot_v, shared) = refs
    else:
      (ei_hbm, out_hbm,
       acc_v, src_v, dst_v, tmp_v, tot_v, shared) = refs
    cid = lax.axis_index("c")
    sid = lax.axis_index("s")
    base = (cid * NS + sid) * e_w
    if vals is not None:
      pltpu.sync_copy(vals_hbm, vals_v)

    zero16 = jnp.zeros((16,), jnp.float32)
    ones16 = jnp.ones((16,), jnp.float32)

    def zero_body(i, _):
      acc_v[pl.ds(i * 16, 16)] = zero16
      return 0
    lax.fori_loop(0, NPAD // 16, zero_body, 0)

    def chunk_body(c, _):
      off = base + c * ch
      pltpu.sync_copy(ei_hbm.at[0, pl.ds(off, ch)], src_v)
      pltpu.sync_copy(ei_hbm.at[1, pl.ds(off, ch)], dst_v)

      def edge_body(j, _):
        for u in range(unroll):
          p = (j * unroll + u) * 16
          d16 = dst_v[pl.ds(p, 16)]
          if vals is not None:
            s16 = src_v[pl.ds(p, 16)]
            v16 = plsc.load_gather(vals_v, [s16])
          else:
            v16 = ones16
          plsc.addupdate_scatter(acc_v, [d16], v16)
        return 0
      lax.fori_loop(0, ch // (16 * unroll), edge_body, 0)
      return 0
    lax.fori_loop(0, n_ch, chunk_body, 0)

    pltpu.sync_copy(acc_v, shared.at[sid])
    plsc.subcore_barrier()

    def zt_body(i, _):
      tot_v[pl.ds(i * 16, 16)] = zero16
      return 0
    lax.fori_loop(0, slc // 16, zt_body, 0)

    for j in range(NS):
      pltpu.sync_copy(shared.at[j, pl.ds(sid * slc, slc)], tmp_v)

      def add_body(i, _):
        tot_v[pl.ds(i * 16, 16)] += tmp_v[pl.ds(i * 16, 16)]
        return 0
      lax.fori_loop(0, slc // 16, add_body, 0)

    pltpu.sync_copy(tot_v, out_hbm.at[cid, pl.ds(sid * slc, slc)])

  if vals is not None:
    return k(vals, ei)
  return k(ei)


def _edge_propagate(g, ei4):
  n, d = g.shape
  _, nw, n_ch, kk = ei4.shape
  slc = NPADE // NS
  mesh = plsc.VectorSubcoreMesh(core_axis_name="c", subcore_axis_name="s")

  @functools.partial(
      pl.kernel,
      mesh=mesh,
      compiler_params=pltpu.CompilerParams(needs_layout_passes=False,
                                           use_tc_tiling_on_sc=False),
      out_type=jax.ShapeDtypeStruct((NC, NPADE, d), jnp.float32),
      scratch_types=[
          pltpu.VMEM((n_ch, kk), jnp.int32),
          pltpu.VMEM((n_ch, kk), jnp.int32),
          [pltpu.VMEM((kk, d), jnp.float32)] * 2,
          pltpu.VMEM_SHARED((NPADE, d), jnp.float32),
          [pltpu.SemaphoreType.DMA] * 2,
      ],
  )
  def k(g_hbm, ei_hbm, out_hbm, src_v, dst_v, rows, s_sh, sems):
    cid = lax.axis_index("c")
    sid = lax.axis_index("s")
    wid = cid * NS + sid
    pltpu.sync_copy(ei_hbm.at[0, wid], src_v)
    pltpu.sync_copy(ei_hbm.at[1, wid], dst_v)

    zero16 = jnp.zeros((16,), jnp.float32)

    def zr_body(i, _):
      rows[0][i // (d // 16), pl.ds((i % (d // 16)) * 16, 16)] = zero16
      return 0
    lax.fori_loop(0, kk * (d // 16), zr_body, 0)
    for z in range(slc // kk):
      pltpu.sync_copy(rows[0], s_sh.at[pl.ds(sid * slc + z * kk, kk)])
    rem = slc % kk
    if rem:
      pltpu.sync_copy(rows[0].at[pl.ds(0, rem)],
                      s_sh.at[pl.ds(sid * slc + (slc // kk) * kk, rem)])
    plsc.subcore_barrier()

    pltpu.async_copy(g_hbm.at[src_v.at[0]], rows[0], sems[0])

    def pair_body(q, _):
      c0 = q * 2
      pltpu.async_copy(g_hbm.at[src_v.at[c0 + 1]], rows[1], sems[1])
      pltpu.make_async_copy(g_hbm.at[src_v.at[c0]], rows[0], sems[0]).wait()
      pltpu.sync_copy(rows[0], s_sh.at[dst_v.at[c0]], add=True)

      @pl.when(c0 + 2 < n_ch)
      def _():
        pltpu.async_copy(g_hbm.at[src_v.at[c0 + 2]], rows[0], sems[0])

      pltpu.make_async_copy(g_hbm.at[src_v.at[c0]], rows[1], sems[1]).wait()
      pltpu.sync_copy(rows[1], s_sh.at[dst_v.at[c0 + 1]], add=True)
      return 0
    lax.fori_loop(0, n_ch // 2, pair_body, 0)

    plsc.subcore_barrier()
    pltpu.sync_copy(s_sh.at[pl.ds(sid * slc, slc)],
                    out_hbm.at[cid, pl.ds(sid * slc, slc)])

  return k(g, ei4)


def _tc_matmul(x, w1):
  n, d_in = x.shape
  h = w1.shape[1]
  r = n

  def body(x_r, w_r, o_r):
    o_r[...] = jnp.dot(x_r[...], w_r[...], preferred_element_type=jnp.float32)

  return pl.pallas_call(
      body,
      grid=(n // r,),
      in_specs=[
          pl.BlockSpec((r, d_in), lambda i: (i, 0)),
          pl.BlockSpec((d_in, h), lambda i: (0, 0)),
      ],
      out_specs=pl.BlockSpec((r, h), lambda i: (i, 0)),
      out_shape=jax.ShapeDtypeStruct((n, h), jnp.float32),
  )(x, w1)


def _tc_scale1(cnt_a, cnt_b, h1):
  n, h = h1.shape
  r = n

  def body(ca_r, cb_r, h_r, d_r, g_r):
    deg = ca_r[...] + cb_r[...] + 1.0
    dv = lax.rsqrt(deg)
    d_r[...] = dv
    g_r[...] = h_r[...] * dv

  return pl.pallas_call(
      body,
      grid=(n // r,),
      in_specs=[
          pl.BlockSpec((r, 1), lambda i: (i, 0)),
          pl.BlockSpec((r, 1), lambda i: (i, 0)),
          pl.BlockSpec((r, h), lambda i: (i, 0)),
      ],
      out_specs=[
          pl.BlockSpec((r, 1), lambda i: (i, 0)),
          pl.BlockSpec((r, h), lambda i: (i, 0)),
      ],
      out_shape=[
          jax.ShapeDtypeStruct((n, 1), jnp.float32),
          jax.ShapeDtypeStruct((n, h), jnp.float32),
      ],
  )(cnt_a, cnt_b, h1)


def _tc_mid(s, g_prev, d, b, w):
  n, h_in = g_prev.shape
  h_out = w.shape[1]
  r = n

  def body(sa_r, sb_r, g_r, d_r, b_r, w_r, o_r):
    hid = jnp.maximum(
        d_r[...] * (sa_r[0] + sb_r[0] + g_r[...]) + b_r[...], 0.0)
    o_r[...] = jnp.dot(hid, w_r[...],
                       preferred_element_type=jnp.float32) * d_r[...]

  return pl.pallas_call(
      body,
      grid=(n // r,),
      in_specs=[
          pl.BlockSpec((1, r, h_in), lambda i: (0, i, 0)),
          pl.BlockSpec((1, r, h_in), lambda i: (1, i, 0)),
          pl.BlockSpec((r, h_in), lambda i: (i, 0)),
          pl.BlockSpec((r, 1), lambda i: (i, 0)),
          pl.BlockSpec((1, h_in), lambda i: (0, 0)),
          pl.BlockSpec((h_in, h_out), lambda i: (0, 0)),
      ],
      out_specs=pl.BlockSpec((r, h_out), lambda i: (i, 0)),
      out_shape=jax.ShapeDtypeStruct((n, h_out), jnp.float32),
  )(s, s, g_prev, d, b, w)


def _tc_final(s_a, s_b, g3, d, b3):
  n = g3.shape[0]

  def body(sa_r, sb_r, g_r, d_r, b_r, o_r):
    o_r[...] = d_r[...] * (sa_r[...] + sb_r[...] + g_r[...]) + b_r[...]

  return pl.pallas_call(
      body,
      grid=(1,),
      in_specs=[pl.BlockSpec((n, 1), lambda i: (0, 0))] * 4
      + [pl.BlockSpec((1, 1), lambda i: (0, 0))],
      out_specs=pl.BlockSpec((n, 1), lambda i: (0, 0)),
      out_shape=jax.ShapeDtypeStruct((n, 1), jnp.float32),
  )(s_a, s_b, g3, d, b3)


def kernel(x, edge_index, W1, b1, W2, b2, W3, b3):
  n = x.shape[0]
  e = edge_index.shape[1]
  kk = 100
  n_ch = e // (NC * NS * kk)
  ei4 = edge_index.reshape(2, NC * NS, n_ch, kk)

  cnt = _scalar_propagate(None, edge_index)
  h1 = _tc_matmul(x, W1)
  cnt_a = cnt[0, :n].reshape(n, 1)
  cnt_b = cnt[1, :n].reshape(n, 1)

  d, g1 = _tc_scale1(cnt_a, cnt_b, h1)
  s1 = _edge_propagate(g1, ei4)
  g2 = _tc_mid(s1, g1, d, b1.reshape(1, -1), W2)
  s2 = _edge_propagate(g2, ei4)
  g3 = _tc_mid(s2, g2, d, b2.reshape(1, -1), W3)
  s3 = _scalar_propagate(g3.reshape(n), edge_index)
  out = _tc_final(s3[0, :n].reshape(n, 1), s3[1, :n].reshape(n, 1),
                  g3, d, b3.reshape(1, 1))
  return out


# speedup vs baseline: 1.2340x; 1.0211x over previous
import functools

import jax
import jax.numpy as jnp
from jax import lax
from jax.experimental import pallas as pl
from jax.experimental.pallas import tpu as pltpu
from jax.experimental.pallas import tpu_sc as plsc

NC = 2
NS = 16
NPAD = 10240
NPADE = 10048


def _scalar_propagate(vals, ei):
  n = vals.shape[0] if vals is not None else 0
  e = ei.shape[1]
  e_w = e // (NC * NS)
  ch = 2000
  n_ch = e_w // ch
  unroll = 5
  slc = NPAD // NS
  mesh = plsc.VectorSubcoreMesh(core_axis_name="c", subcore_axis_name="s")

  vals_scratch = [pltpu.VMEM((n,), jnp.float32)] if vals is not None else []

  @functools.partial(
      pl.kernel,
      mesh=mesh,
      compiler_params=pltpu.CompilerParams(needs_layout_passes=False,
                                           use_tc_tiling_on_sc=False),
      out_type=jax.ShapeDtypeStruct((NC, NPAD), jnp.float32),
      scratch_types=vals_scratch + [
          pltpu.VMEM((NPAD,), jnp.float32),
          pltpu.VMEM((ch,), jnp.int32),
          pltpu.VMEM((ch,), jnp.int32),
          pltpu.VMEM((slc,), jnp.float32),
          pltpu.VMEM((slc,), jnp.float32),
          pltpu.VMEM_SHARED((NS, NPAD), jnp.float32),
      ],
  )
  def k(*refs):
    if vals is not None:
      (vals_hbm, ei_hbm, out_hbm,
       vals_v, acc_v, src_v, dst_v, tmp_v, tot_v, shared) = refs
    else:
      (ei_hbm, out_hbm,
       acc_v, src_v, dst_v, tmp_v, tot_v, shared) = refs
    cid = lax.axis_index("c")
    sid = lax.axis_index("s")
    base = (cid * NS + sid) * e_w
    if vals is not None:
      pltpu.sync_copy(vals_hbm, vals_v)

    zero16 = jnp.zeros((16,), jnp.float32)
    ones16 = jnp.ones((16,), jnp.float32)

    def zero_body(i, _):
      acc_v[pl.ds(i * 16, 16)] = zero16
      return 0
    lax.fori_loop(0, NPAD // 16, zero_body, 0)

    def chunk_body(c, _):
      off = base + c * ch
      pltpu.sync_copy(ei_hbm.at[0, pl.ds(off, ch)], src_v)
      pltpu.sync_copy(ei_hbm.at[1, pl.ds(off, ch)], dst_v)

      @plsc.parallel_loop(0, ch // 16, unroll=unroll)
      def edge_body(j):
        p = j * 16
        d16 = dst_v[pl.ds(p, 16)]
        if vals is not None:
          s16 = src_v[pl.ds(p, 16)]
          v16 = plsc.load_gather(vals_v, [s16])
        else:
          v16 = ones16
        plsc.addupdate_scatter(acc_v, [d16], v16)
      return 0
    lax.fori_loop(0, n_ch, chunk_body, 0)

    pltpu.sync_copy(acc_v, shared.at[sid])
    plsc.subcore_barrier()

    def zt_body(i, _):
      tot_v[pl.ds(i * 16, 16)] = zero16
      return 0
    lax.fori_loop(0, slc // 16, zt_body, 0)

    for j in range(NS):
      pltpu.sync_copy(shared.at[j, pl.ds(sid * slc, slc)], tmp_v)

      def add_body(i, _):
        tot_v[pl.ds(i * 16, 16)] += tmp_v[pl.ds(i * 16, 16)]
        return 0
      lax.fori_loop(0, slc // 16, add_body, 0)

    pltpu.sync_copy(tot_v, out_hbm.at[cid, pl.ds(sid * slc, slc)])

  if vals is not None:
    return k(vals, ei)
  return k(ei)


def _edge_propagate(g, ei4):
  n, d = g.shape
  _, nw, n_ch, kk = ei4.shape
  slc = NPADE // NS
  mesh = plsc.VectorSubcoreMesh(core_axis_name="c", subcore_axis_name="s")

  @functools.partial(
      pl.kernel,
      mesh=mesh,
      compiler_params=pltpu.CompilerParams(needs_layout_passes=False,
                                           use_tc_tiling_on_sc=False),
      out_type=jax.ShapeDtypeStruct((NC, NPADE, d), jnp.float32),
      scratch_types=[
          pltpu.VMEM((n_ch, kk), jnp.int32),
          pltpu.VMEM((n_ch, kk), jnp.int32),
          [pltpu.VMEM((kk, d), jnp.float32)] * 2,
          pltpu.VMEM_SHARED((NPADE, d), jnp.float32),
          [pltpu.SemaphoreType.DMA] * 2,
      ],
  )
  def k(g_hbm, ei_hbm, out_hbm, src_v, dst_v, rows, s_sh, sems):
    cid = lax.axis_index("c")
    sid = lax.axis_index("s")
    wid = cid * NS + sid
    pltpu.sync_copy(ei_hbm.at[0, wid], src_v)
    pltpu.sync_copy(ei_hbm.at[1, wid], dst_v)

    zero16 = jnp.zeros((16,), jnp.float32)

    def zr_body(i, _):
      rows[0][i // (d // 16), pl.ds((i % (d // 16)) * 16, 16)] = zero16
      return 0
    lax.fori_loop(0, kk * (d // 16), zr_body, 0)
    for z in range(slc // kk):
      pltpu.sync_copy(rows[0], s_sh.at[pl.ds(sid * slc + z * kk, kk)])
    rem = slc % kk
    if rem:
      pltpu.sync_copy(rows[0].at[pl.ds(0, rem)],
                      s_sh.at[pl.ds(sid * slc + (slc // kk) * kk, rem)])
    plsc.subcore_barrier()

    pltpu.async_copy(g_hbm.at[src_v.at[0]], rows[0], sems[0])

    def pair_body(q, _):
      c0 = q * 2
      pltpu.async_copy(g_hbm.at[src_v.at[c0 + 1]], rows[1], sems[1])
      pltpu.make_async_copy(g_hbm.at[src_v.at[c0]], rows[0], sems[0]).wait()
      pltpu.sync_copy(rows[0], s_sh.at[dst_v.at[c0]], add=True)

      @pl.when(c0 + 2 < n_ch)
      def _():
        pltpu.async_copy(g_hbm.at[src_v.at[c0 + 2]], rows[0], sems[0])

      pltpu.make_async_copy(g_hbm.at[src_v.at[c0]], rows[1], sems[1]).wait()
      pltpu.sync_copy(rows[1], s_sh.at[dst_v.at[c0 + 1]], add=True)
      return 0
    lax.fori_loop(0, n_ch // 2, pair_body, 0)

    plsc.subcore_barrier()
    pltpu.sync_copy(s_sh.at[pl.ds(sid * slc, slc)],
                    out_hbm.at[cid, pl.ds(sid * slc, slc)])

  return k(g, ei4)


def _tc_matmul(x, w1):
  n, d_in = x.shape
  h = w1.shape[1]
  r = n

  def body(x_r, w_r, o_r):
    o_r[...] = jnp.dot(x_r[...], w_r[...], preferred_element_type=jnp.float32)

  return pl.pallas_call(
      body,
      grid=(n // r,),
      in_specs=[
          pl.BlockSpec((r, d_in), lambda i: (i, 0)),
          pl.BlockSpec((d_in, h), lambda i: (0, 0)),
      ],
      out_specs=pl.BlockSpec((r, h), lambda i: (i, 0)),
      out_shape=jax.ShapeDtypeStruct((n, h), jnp.float32),
  )(x, w1)


def _tc_scale1(cnt_a, cnt_b, h1):
  n, h = h1.shape
  r = n

  def body(ca_r, cb_r, h_r, d_r, g_r):
    deg = ca_r[...] + cb_r[...] + 1.0
    dv = lax.rsqrt(deg)
    d_r[...] = dv
    g_r[...] = h_r[...] * dv

  return pl.pallas_call(
      body,
      grid=(n // r,),
      in_specs=[
          pl.BlockSpec((r, 1), lambda i: (i, 0)),
          pl.BlockSpec((r, 1), lambda i: (i, 0)),
          pl.BlockSpec((r, h), lambda i: (i, 0)),
      ],
      out_specs=[
          pl.BlockSpec((r, 1), lambda i: (i, 0)),
          pl.BlockSpec((r, h), lambda i: (i, 0)),
      ],
      out_shape=[
          jax.ShapeDtypeStruct((n, 1), jnp.float32),
          jax.ShapeDtypeStruct((n, h), jnp.float32),
      ],
  )(cnt_a, cnt_b, h1)


def _tc_mid(s, g_prev, d, b, w):
  n, h_in = g_prev.shape
  h_out = w.shape[1]
  r = n

  def body(sa_r, sb_r, g_r, d_r, b_r, w_r, o_r):
    hid = jnp.maximum(
        d_r[...] * (sa_r[0] + sb_r[0] + g_r[...]) + b_r[...], 0.0)
    o_r[...] = jnp.dot(hid, w_r[...],
                       preferred_element_type=jnp.float32) * d_r[...]

  return pl.pallas_call(
      body,
      grid=(n // r,),
      in_specs=[
          pl.BlockSpec((1, r, h_in), lambda i: (0, i, 0)),
          pl.BlockSpec((1, r, h_in), lambda i: (1, i, 0)),
          pl.BlockSpec((r, h_in), lambda i: (i, 0)),
          pl.BlockSpec((r, 1), lambda i: (i, 0)),
          pl.BlockSpec((1, h_in), lambda i: (0, 0)),
          pl.BlockSpec((h_in, h_out), lambda i: (0, 0)),
      ],
      out_specs=pl.BlockSpec((r, h_out), lambda i: (i, 0)),
      out_shape=jax.ShapeDtypeStruct((n, h_out), jnp.float32),
  )(s, s, g_prev, d, b, w)


def _tc_final(s_a, s_b, g3, d, b3):
  n = g3.shape[0]

  def body(sa_r, sb_r, g_r, d_r, b_r, o_r):
    o_r[...] = d_r[...] * (sa_r[...] + sb_r[...] + g_r[...]) + b_r[...]

  return pl.pallas_call(
      body,
      grid=(1,),
      in_specs=[pl.BlockSpec((n, 1), lambda i: (0, 0))] * 4
      + [pl.BlockSpec((1, 1), lambda i: (0, 0))],
      out_specs=pl.BlockSpec((n, 1), lambda i: (0, 0)),
      out_shape=jax.ShapeDtypeStruct((n, 1), jnp.float32),
  )(s_a, s_b, g3, d, b3)


def kernel(x, edge_index, W1, b1, W2, b2, W3, b3):
  n = x.shape[0]
  e = edge_index.shape[1]
  kk = 100
  n_ch = e // (NC * NS * kk)
  ei4 = edge_index.reshape(2, NC * NS, n_ch, kk)

  cnt = _scalar_propagate(None, edge_index)
  h1 = _tc_matmul(x, W1)
  cnt_a = cnt[0, :n].reshape(n, 1)
  cnt_b = cnt[1, :n].reshape(n, 1)

  d, g1 = _tc_scale1(cnt_a, cnt_b, h1)
  s1 = _edge_propagate(g1, ei4)
  g2 = _tc_mid(s1, g1, d, b1.reshape(1, -1), W2)
  s2 = _edge_propagate(g2, ei4)
  g3 = _tc_mid(s2, g2, d, b2.reshape(1, -1), W3)
  s3 = _scalar_propagate(g3.reshape(n), edge_index)
  out = _tc_final(s3[0, :n].reshape(n, 1), s3[1, :n].reshape(n, 1),
                  g3, d, b3.reshape(1, 1))
  return out


# speedup vs baseline: 1.2494x; 1.0125x over previous
import functools

import jax
import jax.numpy as jnp
from jax import lax
from jax.experimental import pallas as pl
from jax.experimental.pallas import tpu as pltpu
from jax.experimental.pallas import tpu_sc as plsc

NC = 2
NS = 16
NPAD = 10240
NPADE = 10048


def _scalar_propagate(vals, ei):
  n = vals.shape[0] if vals is not None else 0
  e = ei.shape[1]
  e_w = e // (NC * NS)
  ch = e_w
  n_ch = e_w // ch
  unroll = 8
  slc = NPAD // NS
  mesh = plsc.VectorSubcoreMesh(core_axis_name="c", subcore_axis_name="s")

  vals_scratch = [pltpu.VMEM((n,), jnp.float32)] if vals is not None else []

  @functools.partial(
      pl.kernel,
      mesh=mesh,
      compiler_params=pltpu.CompilerParams(needs_layout_passes=False,
                                           use_tc_tiling_on_sc=False),
      out_type=jax.ShapeDtypeStruct((NC, NPAD), jnp.float32),
      scratch_types=vals_scratch + [
          pltpu.VMEM((NPAD,), jnp.float32),
          pltpu.VMEM((ch,), jnp.int32),
          pltpu.VMEM((ch,), jnp.int32),
          pltpu.VMEM((slc,), jnp.float32),
          pltpu.VMEM((slc,), jnp.float32),
          pltpu.VMEM_SHARED((NS, NPAD), jnp.float32),
      ],
  )
  def k(*refs):
    if vals is not None:
      (vals_hbm, ei_hbm, out_hbm,
       vals_v, acc_v, src_v, dst_v, tmp_v, tot_v, shared) = refs
    else:
      (ei_hbm, out_hbm,
       acc_v, src_v, dst_v, tmp_v, tot_v, shared) = refs
    cid = lax.axis_index("c")
    sid = lax.axis_index("s")
    base = (cid * NS + sid) * e_w
    if vals is not None:
      pltpu.sync_copy(vals_hbm, vals_v)

    zero16 = jnp.zeros((16,), jnp.float32)
    ones16 = jnp.ones((16,), jnp.float32)

    def zero_body(i, _):
      acc_v[pl.ds(i * 16, 16)] = zero16
      return 0
    lax.fori_loop(0, NPAD // 16, zero_body, 0)

    def chunk_body(c, _):
      off = base + c * ch
      pltpu.sync_copy(ei_hbm.at[0, pl.ds(off, ch)], src_v)
      pltpu.sync_copy(ei_hbm.at[1, pl.ds(off, ch)], dst_v)

      @plsc.parallel_loop(0, ch // 16, unroll=unroll)
      def edge_body(j):
        p = j * 16
        d16 = dst_v[pl.ds(p, 16)]
        if vals is not None:
          s16 = src_v[pl.ds(p, 16)]
          v16 = plsc.load_gather(vals_v, [s16])
        else:
          v16 = ones16
        plsc.addupdate_scatter(acc_v, [d16], v16)
      return 0
    lax.fori_loop(0, n_ch, chunk_body, 0)

    pltpu.sync_copy(acc_v, shared.at[sid])
    plsc.subcore_barrier()

    def zt_body(i, _):
      tot_v[pl.ds(i * 16, 16)] = zero16
      return 0
    lax.fori_loop(0, slc // 16, zt_body, 0)

    for j in range(NS):
      pltpu.sync_copy(shared.at[j, pl.ds(sid * slc, slc)], tmp_v)

      def add_body(i, _):
        tot_v[pl.ds(i * 16, 16)] += tmp_v[pl.ds(i * 16, 16)]
        return 0
      lax.fori_loop(0, slc // 16, add_body, 0)

    pltpu.sync_copy(tot_v, out_hbm.at[cid, pl.ds(sid * slc, slc)])

  if vals is not None:
    return k(vals, ei)
  return k(ei)


def _edge_propagate(g, ei4):
  n, d = g.shape
  _, nw, n_ch, kk = ei4.shape
  slc = NPADE // NS
  mesh = plsc.VectorSubcoreMesh(core_axis_name="c", subcore_axis_name="s")

  @functools.partial(
      pl.kernel,
      mesh=mesh,
      compiler_params=pltpu.CompilerParams(needs_layout_passes=False,
                                           use_tc_tiling_on_sc=False),
      out_type=jax.ShapeDtypeStruct((NC, NPADE, d), jnp.float32),
      scratch_types=[
          pltpu.VMEM((n_ch, kk), jnp.int32),
          pltpu.VMEM((n_ch, kk), jnp.int32),
          [pltpu.VMEM((kk, d), jnp.float32)] * 2,
          pltpu.VMEM_SHARED((NPADE, d), jnp.float32),
          [pltpu.SemaphoreType.DMA] * 2,
      ],
  )
  def k(g_hbm, ei_hbm, out_hbm, src_v, dst_v, rows, s_sh, sems):
    cid = lax.axis_index("c")
    sid = lax.axis_index("s")
    wid = cid * NS + sid
    pltpu.sync_copy(ei_hbm.at[0, wid], src_v)
    pltpu.sync_copy(ei_hbm.at[1, wid], dst_v)

    zero16 = jnp.zeros((16,), jnp.float32)

    def zr_body(i, _):
      rows[0][i // (d // 16), pl.ds((i % (d // 16)) * 16, 16)] = zero16
      return 0
    lax.fori_loop(0, kk * (d // 16), zr_body, 0)
    for z in range(slc // kk):
      pltpu.sync_copy(rows[0], s_sh.at[pl.ds(sid * slc + z * kk, kk)])
    rem = slc % kk
    if rem:
      pltpu.sync_copy(rows[0].at[pl.ds(0, rem)],
                      s_sh.at[pl.ds(sid * slc + (slc // kk) * kk, rem)])
    plsc.subcore_barrier()

    pltpu.async_copy(g_hbm.at[src_v.at[0]], rows[0], sems[0])

    def pair_body(q, _):
      c0 = q * 2
      pltpu.async_copy(g_hbm.at[src_v.at[c0 + 1]], rows[1], sems[1])
      pltpu.make_async_copy(g_hbm.at[src_v.at[c0]], rows[0], sems[0]).wait()
      pltpu.sync_copy(rows[0], s_sh.at[dst_v.at[c0]], add=True)

      @pl.when(c0 + 2 < n_ch)
      def _():
        pltpu.async_copy(g_hbm.at[src_v.at[c0 + 2]], rows[0], sems[0])

      pltpu.make_async_copy(g_hbm.at[src_v.at[c0]], rows[1], sems[1]).wait()
      pltpu.sync_copy(rows[1], s_sh.at[dst_v.at[c0 + 1]], add=True)
      return 0
    lax.fori_loop(0, n_ch // 2, pair_body, 0)

    plsc.subcore_barrier()
    pltpu.sync_copy(s_sh.at[pl.ds(sid * slc, slc)],
                    out_hbm.at[cid, pl.ds(sid * slc, slc)])

  return k(g, ei4)


def _tc_matmul(x, w1):
  n, d_in = x.shape
  h = w1.shape[1]
  r = n

  def body(x_r, w_r, o_r):
    o_r[...] = jnp.dot(x_r[...], w_r[...], preferred_element_type=jnp.float32)

  return pl.pallas_call(
      body,
      grid=(n // r,),
      in_specs=[
          pl.BlockSpec((r, d_in), lambda i: (i, 0)),
          pl.BlockSpec((d_in, h), lambda i: (0, 0)),
      ],
      out_specs=pl.BlockSpec((r, h), lambda i: (i, 0)),
      out_shape=jax.ShapeDtypeStruct((n, h), jnp.float32),
  )(x, w1)


def _tc_scale1(cnt_a, cnt_b, h1):
  n, h = h1.shape
  r = n

  def body(ca_r, cb_r, h_r, d_r, g_r):
    deg = ca_r[...] + cb_r[...] + 1.0
    dv = lax.rsqrt(deg)
    d_r[...] = dv
    g_r[...] = h_r[...] * dv

  return pl.pallas_call(
      body,
      grid=(n // r,),
      in_specs=[
          pl.BlockSpec((r, 1), lambda i: (i, 0)),
          pl.BlockSpec((r, 1), lambda i: (i, 0)),
          pl.BlockSpec((r, h), lambda i: (i, 0)),
      ],
      out_specs=[
          pl.BlockSpec((r, 1), lambda i: (i, 0)),
          pl.BlockSpec((r, h), lambda i: (i, 0)),
      ],
      out_shape=[
          jax.ShapeDtypeStruct((n, 1), jnp.float32),
          jax.ShapeDtypeStruct((n, h), jnp.float32),
      ],
  )(cnt_a, cnt_b, h1)


def _tc_mid(s, g_prev, d, b, w):
  n, h_in = g_prev.shape
  h_out = w.shape[1]
  r = n

  def body(sa_r, sb_r, g_r, d_r, b_r, w_r, o_r):
    hid = jnp.maximum(
        d_r[...] * (sa_r[0] + sb_r[0] + g_r[...]) + b_r[...], 0.0)
    o_r[...] = jnp.dot(hid, w_r[...],
                       preferred_element_type=jnp.float32) * d_r[...]

  return pl.pallas_call(
      body,
      grid=(n // r,),
      in_specs=[
          pl.BlockSpec((1, r, h_in), lambda i: (0, i, 0)),
          pl.BlockSpec((1, r, h_in), lambda i: (1, i, 0)),
          pl.BlockSpec((r, h_in), lambda i: (i, 0)),
          pl.BlockSpec((r, 1), lambda i: (i, 0)),
          pl.BlockSpec((1, h_in), lambda i: (0, 0)),
          pl.BlockSpec((h_in, h_out), lambda i: (0, 0)),
      ],
      out_specs=pl.BlockSpec((r, h_out), lambda i: (i, 0)),
      out_shape=jax.ShapeDtypeStruct((n, h_out), jnp.float32),
  )(s, s, g_prev, d, b, w)


def _tc_final(s_a, s_b, g3, d, b3):
  n = g3.shape[0]

  def body(sa_r, sb_r, g_r, d_r, b_r, o_r):
    o_r[...] = d_r[...] * (sa_r[...] + sb_r[...] + g_r[...]) + b_r[...]

  return pl.pallas_call(
      body,
      grid=(1,),
      in_specs=[pl.BlockSpec((n, 1), lambda i: (0, 0))] * 4
      + [pl.BlockSpec((1, 1), lambda i: (0, 0))],
      out_specs=pl.BlockSpec((n, 1), lambda i: (0, 0)),
      out_shape=jax.ShapeDtypeStruct((n, 1), jnp.float32),
  )(s_a, s_b, g3, d, b3)


def kernel(x, edge_index, W1, b1, W2, b2, W3, b3):
  n = x.shape[0]
  e = edge_index.shape[1]
  kk = 100
  n_ch = e // (NC * NS * kk)
  ei4 = edge_index.reshape(2, NC * NS, n_ch, kk)

  cnt = _scalar_propagate(None, edge_index)
  h1 = _tc_matmul(x, W1)
  cnt_a = cnt[0, :n].reshape(n, 1)
  cnt_b = cnt[1, :n].reshape(n, 1)

  d, g1 = _tc_scale1(cnt_a, cnt_b, h1)
  s1 = _edge_propagate(g1, ei4)
  g2 = _tc_mid(s1, g1, d, b1.reshape(1, -1), W2)
  s2 = _edge_propagate(g2, ei4)
  g3 = _tc_mid(s2, g2, d, b2.reshape(1, -1), W3)
  s3 = _scalar_propagate(g3.reshape(n), edge_index)
  out = _tc_final(s3[0, :n].reshape(n, 1), s3[1, :n].reshape(n, 1),
                  g3, d, b3.reshape(1, 1))
  return out
